# baseline (speedup 1.0000x reference)
import functools

import jax
import jax.numpy as jnp
from jax import lax
from jax.experimental import pallas as pl
from jax.experimental.pallas import tpu as pltpu
from jax.experimental.pallas import tpu_sc as plsc

_VOCAB = 1000000
_EMB = 64
_SEQ = 20
_BATCH = 16384
_TARGET = 16

_NC = 2
_NS = 16
_NW = _NC * _NS

_N_ROWS = _BATCH * _SEQ
_G = 128
_N_CHUNKS = _N_ROWS // _G
_CHUNKS_PER_W = _N_CHUNKS // _NW
_NBUF = 8
_OUT_ROWS = _N_ROWS // 2


def _gather_body(idx_hbm, table_hbm, out_hbm, idx_v, rows_v, gsem):
    wid = lax.axis_index("s") * _NC + lax.axis_index("c")
    c0 = wid * _CHUNKS_PER_W
    pltpu.sync_copy(idx_hbm.at[pl.ds(c0, _CHUNKS_PER_W)], idx_v)

    for k in range(_NBUF):
        pltpu.async_copy(table_hbm.at[idx_v.at[k]], rows_v.at[k], gsem)

    def outer(jj, carry):
        for k in range(_NBUF):
            g = jj * _NBUF + k
            pltpu.make_async_copy(
                table_hbm.at[pl.ds(0, _G)], rows_v.at[k], gsem
            ).wait()
            r0 = (c0 + g) * (_G // 2)
            pltpu.sync_copy(
                rows_v.at[k].at[pl.ds(0, _G // 2)],
                out_hbm.at[pl.ds(r0, _G // 2), pl.ds(0, _EMB)],
            )
            pltpu.sync_copy(
                rows_v.at[k].at[pl.ds(_G // 2, _G // 2)],
                out_hbm.at[pl.ds(r0, _G // 2), pl.ds(_EMB, _EMB)],
            )

            @pl.when(g + _NBUF < _CHUNKS_PER_W)
            def _():
                pltpu.async_copy(
                    table_hbm.at[idx_v.at[g + _NBUF]], rows_v.at[k], gsem
                )

        return carry

    lax.fori_loop(0, _CHUNKS_PER_W // _NBUF, outer, 0, unroll=False)


_gather_call = functools.partial(
    pl.kernel,
    out_type=jax.ShapeDtypeStruct((_OUT_ROWS, 2 * _EMB), jnp.float32),
    mesh=plsc.VectorSubcoreMesh(core_axis_name="c", subcore_axis_name="s"),
    scratch_types=[
        pltpu.VMEM((_CHUNKS_PER_W, _G), jnp.int32),
        pltpu.VMEM((_NBUF, _G, _EMB), jnp.float32),
        pltpu.SemaphoreType.DMA,
    ],
    compiler_params=pltpu.CompilerParams(use_tc_tiling_on_sc=False),
)(_gather_body)


_NTC = 7813
_TAIL = _VOCAB - 128 * (_NTC - 1)


_TBC = 24576


def _tr_body(t_ref, o_ref):
    t2 = jnp.swapaxes(t_ref[...], 0, 1)
    o_ref[...] = jnp.concatenate([t2, t2], axis=1)


def _relayout_call(tabt):
    return pl.pallas_call(
        _tr_body,
        grid=((_VOCAB + _TBC - 1) // _TBC,),
        in_specs=[pl.BlockSpec((_EMB, _TBC), lambda i: (0, i))],
        out_specs=pl.BlockSpec((_TBC, 2 * _EMB), lambda i: (i, 0)),
        out_shape=jax.ShapeDtypeStruct((_VOCAB, 2 * _EMB), jnp.float32),
    )(tabt)


_BM = 2048
_NJ = _SEQ // 2


def _mm_body(f_ref, w_ref, b_ref, o_ref):
    acc = b_ref[...]
    for j in range(_NJ):
        f = jnp.maximum(f_ref[j], 0.0)
        acc = acc + jnp.dot(
            f,
            w_ref[pl.ds(j * 2 * _EMB, 2 * _EMB), :],
            preferred_element_type=jnp.float32,
        )
    o_ref[...] = acc


def _mm_call(feats3, W, b2):
    return pl.pallas_call(
        _mm_body,
        grid=(_BATCH // _BM,),
        in_specs=[
            pl.BlockSpec((_NJ, _BM, 2 * _EMB), lambda i: (0, i, 0)),
            pl.BlockSpec((_SEQ * _EMB, _TARGET), lambda i: (0, 0)),
            pl.BlockSpec((1, _TARGET), lambda i: (0, 0)),
        ],
        out_specs=pl.BlockSpec((_BM, _TARGET), lambda i: (i, 0)),
        out_shape=jax.ShapeDtypeStruct((_BATCH, _TARGET), jnp.float32),
    )(feats3, W, b2)


def kernel(x, table, W, b):
    xp = (
        x.astype(jnp.int32)
        .reshape(_BATCH, _NJ, 2)
        .transpose(1, 0, 2)
        .reshape(_NJ, _BATCH // 64, 64, 2)
        .transpose(0, 1, 3, 2)
        .reshape(_N_CHUNKS, _G)
    )
    table128 = _relayout_call(table.T)
    table64 = table128.reshape(2 * _VOCAB, _EMB)
    feats = _gather_call(xp * 2, table64)
    feats3 = feats.reshape(_NJ, _BATCH, 2 * _EMB)
    return _mm_call(feats3, W, b.reshape(1, _TARGET))


# speedup vs baseline: 1.0057x; 1.0057x over previous
import functools

import jax
import jax.numpy as jnp
from jax import lax
from jax.experimental import pallas as pl
from jax.experimental.pallas import tpu as pltpu
from jax.experimental.pallas import tpu_sc as plsc

_VOCAB = 1000000
_EMB = 64
_SEQ = 20
_BATCH = 16384
_TARGET = 16

_NC = 2
_NS = 16
_NW = _NC * _NS

_N_ROWS = _BATCH * _SEQ
_G = 128
_N_CHUNKS = _N_ROWS // _G
_CHUNKS_PER_W = _N_CHUNKS // _NW
_NBUF = 4
_OUT_ROWS = _N_ROWS // 2


def _gather_body(idx_hbm, table_hbm, out_hbm, idx_v, rows_v, gsem):
    wid = lax.axis_index("s") * _NC + lax.axis_index("c")
    c0 = wid * _CHUNKS_PER_W
    pltpu.sync_copy(idx_hbm.at[pl.ds(c0, _CHUNKS_PER_W)], idx_v)

    for k in range(_NBUF):
        pltpu.async_copy(table_hbm.at[idx_v.at[k]], rows_v.at[k], gsem)

    def outer(jj, carry):
        for k in range(_NBUF):
            g = jj * _NBUF + k
            pltpu.make_async_copy(
                table_hbm.at[pl.ds(0, _G)], rows_v.at[k], gsem
            ).wait()
            r0 = (c0 + g) * (_G // 2)
            pltpu.sync_copy(
                rows_v.at[k].at[pl.ds(0, _G // 2)],
                out_hbm.at[pl.ds(r0, _G // 2), pl.ds(0, _EMB)],
            )
            pltpu.sync_copy(
                rows_v.at[k].at[pl.ds(_G // 2, _G // 2)],
                out_hbm.at[pl.ds(r0, _G // 2), pl.ds(_EMB, _EMB)],
            )

            @pl.when(g + _NBUF < _CHUNKS_PER_W)
            def _():
                pltpu.async_copy(
                    table_hbm.at[idx_v.at[g + _NBUF]], rows_v.at[k], gsem
                )

        return carry

    lax.fori_loop(0, _CHUNKS_PER_W // _NBUF, outer, 0, unroll=False)


_gather_call = functools.partial(
    pl.kernel,
    out_type=jax.ShapeDtypeStruct((_OUT_ROWS, 2 * _EMB), jnp.float32),
    mesh=plsc.VectorSubcoreMesh(core_axis_name="c", subcore_axis_name="s"),
    scratch_types=[
        pltpu.VMEM((_CHUNKS_PER_W, _G), jnp.int32),
        pltpu.VMEM((_NBUF, _G, _EMB), jnp.float32),
        pltpu.SemaphoreType.DMA,
    ],
    compiler_params=pltpu.CompilerParams(use_tc_tiling_on_sc=False),
)(_gather_body)


_NTC = 7813
_TAIL = _VOCAB - 128 * (_NTC - 1)


_TBC = 24576


def _tr_body(t_ref, o_ref):
    t2 = jnp.swapaxes(t_ref[...], 0, 1)
    o_ref[...] = jnp.concatenate([t2, t2], axis=1)


def _relayout_call(tabt):
    return pl.pallas_call(
        _tr_body,
        grid=((_VOCAB + _TBC - 1) // _TBC,),
        in_specs=[pl.BlockSpec((_EMB, _TBC), lambda i: (0, i))],
        out_specs=pl.BlockSpec((_TBC, 2 * _EMB), lambda i: (i, 0)),
        out_shape=jax.ShapeDtypeStruct((_VOCAB, 2 * _EMB), jnp.float32),
    )(tabt)


_BM = 2048
_NJ = _SEQ // 2


def _mm_body(f_ref, w_ref, b_ref, o_ref):
    acc = b_ref[...]
    for j in range(_NJ):
        f = jnp.maximum(f_ref[j], 0.0)
        acc = acc + jnp.dot(
            f,
            w_ref[pl.ds(j * 2 * _EMB, 2 * _EMB), :],
            preferred_element_type=jnp.float32,
        )
    o_ref[...] = acc


def _mm_call(feats3, W, b2):
    return pl.pallas_call(
        _mm_body,
        grid=(_BATCH // _BM,),
        in_specs=[
            pl.BlockSpec((_NJ, _BM, 2 * _EMB), lambda i: (0, i, 0)),
            pl.BlockSpec((_SEQ * _EMB, _TARGET), lambda i: (0, 0)),
            pl.BlockSpec((1, _TARGET), lambda i: (0, 0)),
        ],
        out_specs=pl.BlockSpec((_BM, _TARGET), lambda i: (i, 0)),
        out_shape=jax.ShapeDtypeStruct((_BATCH, _TARGET), jnp.float32),
    )(feats3, W, b2)


def kernel(x, table, W, b):
    xp = (
        x.astype(jnp.int32)
        .reshape(_BATCH, _NJ, 2)
        .transpose(1, 0, 2)
        .reshape(_NJ, _BATCH // 64, 64, 2)
        .transpose(0, 1, 3, 2)
        .reshape(_N_CHUNKS, _G)
    )
    table128 = _relayout_call(table.T)
    table64 = table128.reshape(2 * _VOCAB, _EMB)
    feats = _gather_call(xp * 2, table64)
    feats3 = feats.reshape(_NJ, _BATCH, 2 * _EMB)
    return _mm_call(feats3, W, b.reshape(1, _TARGET))
